# baseline (speedup 1.0000x reference)
import functools

import jax
import jax.numpy as jnp
import numpy as np
from jax import lax
from jax.experimental import pallas as pl
from jax.experimental.pallas import tpu as pltpu
from jax.experimental.pallas import tpu_sc as plsc

_B = 4096
_D = 128
_NW = 32
_RPW = _B // _NW

_SC_FEATS = (
    (0, 16, 8),
    (6, 5, 16),
    (7, 5, 16),
    (8, 5, 16),
    (9, 5, 16),
    (10, 5, 16),
    (11, 5, 16),
    (12, 5, 16),
    (13, 5, 16),
)

_TC_FEATS = (
    (1, 4, 1),
    (2, 21, 1),
    (3, 21, 1),
    (4, 21, 1),
    (5, 21, 1),
    (14, 21, 5),
    (15, 21, 5),
    (16, 21, 5),
    (17, 21, 5),
    (18, 21, 5),
    (19, 21, 5),
    (20, 13, 5),
)

_NSC = len(_SC_FEATS)


def _sc_gather_pool(tables, flat_ids, nb):
  rpw = nb // _NW
  sub5 = rpw // 8
  sub16 = rpw // 16
  cnt5 = 5 * sub5
  cnt16 = 16 * sub16
  subs = {5: sub5, 16: sub16}
  mesh = plsc.VectorSubcoreMesh(core_axis_name="c", subcore_axis_name="s")
  out_type = jax.ShapeDtypeStruct((nb, _NSC * _D), jnp.float32)
  scratch = [
      pltpu.VMEM((8, cnt5), jnp.int32),
      pltpu.VMEM((8, cnt5), jnp.int32),
      pltpu.VMEM((16, cnt16), jnp.int32),
      pltpu.VMEM((cnt5, _D), jnp.float32),
      pltpu.VMEM((cnt5, _D), jnp.float32),
      pltpu.VMEM((cnt5, _D), jnp.float32),
      pltpu.VMEM((cnt5, _D), jnp.float32),
      pltpu.VMEM((cnt16, _D), jnp.float32),
      pltpu.VMEM((cnt16, _D), jnp.float32),
      pltpu.VMEM((rpw, _D), jnp.float32),
      pltpu.VMEM((rpw, _D), jnp.float32),
      pltpu.SemaphoreType.DMA,
      pltpu.SemaphoreType.DMA,
      pltpu.SemaphoreType.DMA,
      pltpu.SemaphoreType.DMA,
      pltpu.SemaphoreType.DMA,
      pltpu.SemaphoreType.DMA,
      pltpu.SemaphoreType.DMA,
      pltpu.SemaphoreType.DMA,
  ]

  @functools.partial(pl.kernel, out_type=out_type, mesh=mesh,
                     scratch_types=scratch)
  def k(*refs):
    ids = refs[0:_NSC]
    tabs = refs[_NSC:2 * _NSC]
    out = refs[2 * _NSC]
    (idx5a, idx5b, idx16, r5a, r5b, r5c, r5d, r16a, r16b, acca, accb,
     g0, g1, g2, g3, g4, g5, oa, ob) = refs[2 * _NSC + 1:]
    wid = lax.axis_index("s") * 2 + lax.axis_index("c")
    base = wid * rpw
    accs = (acca, accb)
    osems = (oa, ob)
    idx5s = (idx5a, idx5b)

    def cfg(fi):
      L = _SC_FEATS[fi][1]
      if L == 16:
        return idx16, (r16a, r16b), (g4, g5), 2, subs[16]
      return (idx5s[(fi - 1) % 2], (r5a, r5b, r5c, r5d), (g0, g1, g2, g3),
              4, subs[5])

    def idx_copy(fi):
      idx_v, _, _, _, SUB = cfg(fi)
      nsub = rpw // SUB
      chunk0 = pl.multiple_of(base // SUB, 8)
      pltpu.sync_copy(ids[fi].at[pl.ds(chunk0, nsub)], idx_v)

    def fire(fi, c, p):
      idx_v, rows, gsems, _, _ = cfg(fi)
      pltpu.async_copy(tabs[fi].at[idx_v.at[c]], rows[p], gsems[p])

    def drain_and_pool(fi, c, p, acc_v):
      idx_v, rows, gsems, _, SUB = cfg(fi)
      L = _SC_FEATS[fi][1]
      scale = 1.0 if L == 16 else 1.0 / L
      pltpu.make_async_copy(tabs[fi].at[idx_v.at[c]], rows[p],
                            gsems[p]).wait()

      def acc_body(b, c2, c=c, L=L, scale=scale, p=p, rows=rows,
                   acc_v=acc_v, SUB=SUB):
        for j in range(_D // 16):
          v = rows[p][b * L, pl.ds(j * 16, 16)]
          for l in range(1, L):
            v = v + rows[p][b * L + l, pl.ds(j * 16, 16)]
          acc_v[c * SUB + b, pl.ds(j * 16, 16)] = v * scale
        return c2

      lax.fori_loop(0, SUB, acc_body, 0)

    idx_copy(0)
    _, _, _, nbuf0, _ = cfg(0)
    for c in range(nbuf0 - 1):
      fire(0, c, c)

    for fi in range(_NSC):
      _, L, _ = _SC_FEATS[fi]
      idx_v, rows, gsems, nbuf, SUB = cfg(fi)
      nsub = rpw // SUB
      acc_v = accs[fi % 2]

      if fi >= 2:
        pltpu.make_async_copy(
            acc_v, out.at[pl.ds(base, rpw), pl.ds((fi - 2) * _D, _D)],
            osems[fi % 2]).wait()

      next_cross = fi + 1 < _NSC and _SC_FEATS[fi + 1][1] != L
      nlast = nsub // nbuf - 1

      def step_body(t, carry, fi=fi, nbuf=nbuf, nsub=nsub,
                    next_cross=next_cross, nlast=nlast):
        last = t == nlast
        for kk in range(nbuf):
          c = nbuf * t + kk

          @pl.when(c + nbuf - 1 < nsub)
          def _():
            fire(fi, c + nbuf - 1, (kk + nbuf - 1) % nbuf)

          if kk == 0 and fi + 1 < _NSC:
            @pl.when(last)
            def _():
              idx_copy(fi + 1)
              if next_cross:
                nbuf_n = cfg(fi + 1)[3]
                for cn in range(nbuf_n - 1):
                  fire(fi + 1, cn, cn)

          drain_and_pool(fi, c, kk, accs[fi % 2])

          if fi + 1 < _NSC and not next_cross and kk < nbuf - 1:
            @pl.when(last)
            def _():
              fire(fi + 1, kk, kk)
        return carry

      lax.fori_loop(0, nsub // nbuf, step_body, 0)
      pltpu.async_copy(
          acc_v, out.at[pl.ds(base, rpw), pl.ds(fi * _D, _D)],
          osems[fi % 2])

    for fi in (_NSC - 2, _NSC - 1):
      pltpu.make_async_copy(
          accs[fi % 2], out.at[pl.ds(base, rpw), pl.ds(fi * _D, _D)],
          osems[fi % 2]).wait()

  return k(*flat_ids, *tables)


_VTOT = sum(v for _, v, _ in _TC_FEATS)
_N5 = [s for s, (_, _, L) in enumerate(_TC_FEATS) if L == 5]


def _tc_body(refs, n_small):
  i = 0
  xsc_ref = refs[i]; i += 1
  pn_ids_ref = refs[i]; i += 1
  t0_ref = refs[i]; i += 1
  a0_ref = refs[i]; i += 1
  a1_ref = refs[i]; i += 1
  p0_ref = refs[i]; i += 1
  p7_ref = refs[i]; i += 1
  iota_ref = refs[i]; i += 1
  iota5_ref = refs[i]; i += 1
  scale_ref = refs[i]; i += 1
  small_tab_refs = refs[i:i + n_small]; i += n_small
  w0_ref, b0_ref, w1_ref, b1_ref, w2_ref, b2_ref = refs[i:i + 6]
  out_ref = refs[i + 6]
  wf_ref = refs[i + 7]

  xsc = xsc_ref[...]
  pn_ids = pn_ids_ref[...]
  n0 = jnp.sum((pn_ids == 0).astype(jnp.float32), axis=1, keepdims=True)
  x0 = (xsc[:, 0:_D] - n0 * t0_ref[...]) / jnp.maximum(16.0 - n0, 1.0)

  iotar = iota_ref[...]
  iota5 = iota5_ref[...]
  b0c = jnp.dot(a0_ref[...], p0_ref[...],
                preferred_element_type=jnp.float32)
  cnt = (b0c == iotar).astype(jnp.float32)
  a1 = a1_ref[...]
  for l in range(4):
    blc = jnp.dot(a1[:, l * 7:(l + 1) * 7], p7_ref[...],
                  preferred_element_type=jnp.float32)
    cnt = cnt + (blc == iota5).astype(jnp.float32)
  cnt = cnt * scale_ref[...]

  @pl.when(pl.program_id(0) == 0)
  def _():
    wf_parts = []
    for s, (fi, vocab, L) in enumerate(_TC_FEATS):
      wf_parts.append(jnp.dot(small_tab_refs[s][...],
                              w0_ref[fi * _D:(fi + 1) * _D, :],
                              preferred_element_type=jnp.float32))
    wf_ref[...] = jnp.concatenate(wf_parts, axis=0)

  wf = wf_ref[...]

  bf = jnp.bfloat16
  h = (jnp.dot(x0.astype(bf), w0_ref[0:_D, :].astype(bf),
               preferred_element_type=jnp.float32)
       + jnp.dot(xsc[:, _D:].astype(bf),
                 w0_ref[6 * _D:14 * _D, :].astype(bf),
                 preferred_element_type=jnp.float32)
       + jnp.dot(cnt.astype(bf), wf.astype(bf),
                 preferred_element_type=jnp.float32))
  h = jnp.maximum(h + b0_ref[...], 0.0)
  h = jnp.dot(h.astype(bf), w1_ref[...].astype(bf),
              preferred_element_type=jnp.float32)
  h = jnp.maximum(h + b1_ref[...], 0.0)
  h = jnp.dot(h.astype(bf), w2_ref[...].astype(bf),
              preferred_element_type=jnp.float32)
  out_ref[...] = h + b2_ref[...]


def _tc_mlp(xsc, pn_ids, t0, a0, a1, p0, p7, iota_row, iota5_row,
            scale_row, small_tabs, weights, interpret=False):
  n_small = len(small_tabs)
  nb = xsc.shape[0]
  rows = 512
  grid = (nb // rows,)

  def rowblk(cols):
    return pl.BlockSpec((rows, cols), lambda i: (i, 0))

  def full(a):
    return pl.BlockSpec(a.shape, lambda i: (0, 0))

  in_specs = (
      [rowblk(_NSC * _D)]
      + [rowblk(16)]
      + [full(t0)]
      + [rowblk(12), rowblk(28), full(p0), full(p7), full(iota_row),
         full(iota5_row), full(scale_row)]
      + [full(t) for t in small_tabs]
      + [full(w) for w in weights]
  )
  out_spec = pl.BlockSpec((rows, _D), lambda i: (i, 0))

  f = pl.pallas_call(
      lambda *refs: _tc_body(refs, n_small),
      grid=grid,
      in_specs=in_specs,
      out_specs=out_spec,
      out_shape=jax.ShapeDtypeStruct((nb, _D), jnp.float32),
      scratch_shapes=[pltpu.VMEM((_VTOT, 512), jnp.float32)],
      interpret=interpret,
  )
  return f(xsc, pn_ids, t0, a0, a1, p0, p7, iota_row, iota5_row,
           scale_row, *small_tabs, *weights)


_FEAT_NAMES = (
    "pl_name", "pl_collab", "pl_duration", "num_pl_songs", "num_pl_artists",
    "num_pl_albums", "track_uri", "track_name", "artist_uri", "artist_name",
    "album_uri", "album_name", "artist_genres", "pl_titles", "duration_songs",
    "track_pop", "artist_pop", "artists_followers", "track_danceability",
    "track_energy", "track_key",
)


def kernel(params, pl_name_ids, pl_collab_ids, pl_duration_ids,
           num_pl_songs_ids, num_pl_artists_ids, num_pl_albums_ids,
           track_uri_ids, track_name_ids, artist_uri_ids, artist_name_ids,
           album_uri_ids, album_name_ids, artist_genres_ids, pl_titles_ids,
           duration_songs_ids, track_pop_ids, artist_pop_ids,
           artists_followers_ids, track_danceability_ids, track_energy_ids,
           track_key_ids):
  ids_by_index = (
      pl_name_ids, pl_collab_ids, pl_duration_ids, num_pl_songs_ids,
      num_pl_artists_ids, num_pl_albums_ids, track_uri_ids, track_name_ids,
      artist_uri_ids, artist_name_ids, album_uri_ids, album_name_ids,
      artist_genres_ids, pl_titles_ids, duration_songs_ids, track_pop_ids,
      artist_pop_ids, artists_followers_ids, track_danceability_ids,
      track_energy_ids, track_key_ids,
  )

  sc_tabs = [params["t_" + _FEAT_NAMES[fi]] for fi, _, _ in _SC_FEATS]

  _NH = 1
  nb = _B // _NH
  rpw = nb // _NW
  sc_half_ids = []
  for h in range(_NH):
    lo = h * nb
    ids_h = []
    for fi, L, _ in _SC_FEATS:
      cnt = L * (rpw // 8 if L == 5 else rpw // 16)
      ids_h.append(ids_by_index[fi][lo:lo + nb].reshape(-1, cnt))
    sc_half_ids.append(ids_h)
  sc_feats_halves = [_sc_gather_pool(sc_tabs, sc_half_ids[h], nb)
                     for h in range(_NH)]

  small_ids = [ids_by_index[fi].reshape(_B, -1) for fi, _, _ in _TC_FEATS]
  small_tabs = [params["t_" + _FEAT_NAMES[fi]] for fi, _, _ in _TC_FEATS]
  t0 = params["t_pl_name"][0:1]

  a0 = jnp.stack([ids.astype(jnp.float32)[:, 0] for ids in small_ids],
                 axis=1)
  a1 = jnp.concatenate(
      [jnp.stack([small_ids[s].astype(jnp.float32)[:, l] for s in _N5],
                 axis=1) for l in range(1, 5)], axis=1)
  offs = np.cumsum([0] + [v for _, v, _ in _TC_FEATS])
  p0 = np.zeros((12, _VTOT), np.float32)
  for s in range(len(_TC_FEATS)):
    p0[s, offs[s]:offs[s + 1]] = 1.0
  p7 = np.zeros((7, _VTOT), np.float32)
  for j, s in enumerate(_N5):
    p7[j, offs[s]:offs[s + 1]] = 1.0
  iota_row = np.concatenate(
      [np.arange(v, dtype=np.float32) for _, v, _ in _TC_FEATS])[None]
  iota5_row = np.concatenate(
      [np.arange(v, dtype=np.float32) if L == 5 else np.full(v, -1.0,
                                                             np.float32)
       for _, v, L in _TC_FEATS])[None]
  scale_row = np.concatenate(
      [np.full(v, 1.0 if L == 1 else 1.0 / L, np.float32)
       for _, v, L in _TC_FEATS])[None]

  weights = [
      params["W0"], params["b0"].reshape(1, -1),
      params["W1"], params["b1"].reshape(1, -1),
      params["W2"], params["b2"].reshape(1, -1),
  ]
  outs = []
  for h in range(_NH):
    lo = h * nb
    outs.append(_tc_mlp(
        sc_feats_halves[h], pl_name_ids[lo:lo + nb], t0,
        a0[lo:lo + nb], a1[lo:lo + nb], jnp.asarray(p0), jnp.asarray(p7),
        jnp.asarray(iota_row), jnp.asarray(iota5_row),
        jnp.asarray(scale_row), small_tabs, weights))
  return jnp.concatenate(outs, axis=0)


# speedup vs baseline: 1.0068x; 1.0068x over previous
import functools

import jax
import jax.numpy as jnp
import numpy as np
from jax import lax
from jax.experimental import pallas as pl
from jax.experimental.pallas import tpu as pltpu
from jax.experimental.pallas import tpu_sc as plsc

_B = 4096
_D = 128
_NW = 32
_RPW = _B // _NW

_SC_FEATS = (
    (0, 16, 8),
    (6, 5, 16),
    (7, 5, 16),
    (8, 5, 16),
    (9, 5, 16),
    (10, 5, 16),
    (11, 5, 16),
    (12, 5, 16),
    (13, 5, 16),
)

_TC_FEATS = (
    (1, 4, 1),
    (2, 21, 1),
    (3, 21, 1),
    (4, 21, 1),
    (5, 21, 1),
    (14, 21, 5),
    (15, 21, 5),
    (16, 21, 5),
    (17, 21, 5),
    (18, 21, 5),
    (19, 21, 5),
    (20, 13, 5),
)

_NSC = len(_SC_FEATS)


def _sc_gather_pool(tables, flat_ids, nb):
  rpw = nb // _NW
  sub5 = rpw // 8
  sub16 = rpw // 16
  cnt5 = 5 * sub5
  cnt16 = 16 * sub16
  subs = {5: sub5, 16: sub16}
  mesh = plsc.VectorSubcoreMesh(core_axis_name="c", subcore_axis_name="s")
  out_type = [jax.ShapeDtypeStruct((nb, _D), jnp.float32)
              for _ in range(_NSC)]
  scratch = [
      pltpu.VMEM((8, cnt5), jnp.int32),
      pltpu.VMEM((8, cnt5), jnp.int32),
      pltpu.VMEM((16, cnt16), jnp.int32),
      pltpu.VMEM((cnt5, _D), jnp.float32),
      pltpu.VMEM((cnt5, _D), jnp.float32),
      pltpu.VMEM((cnt5, _D), jnp.float32),
      pltpu.VMEM((cnt5, _D), jnp.float32),
      pltpu.VMEM((cnt16, _D), jnp.float32),
      pltpu.VMEM((cnt16, _D), jnp.float32),
      pltpu.VMEM((rpw, _D), jnp.float32),
      pltpu.VMEM((rpw, _D), jnp.float32),
      pltpu.SemaphoreType.DMA,
      pltpu.SemaphoreType.DMA,
      pltpu.SemaphoreType.DMA,
      pltpu.SemaphoreType.DMA,
      pltpu.SemaphoreType.DMA,
      pltpu.SemaphoreType.DMA,
      pltpu.SemaphoreType.DMA,
      pltpu.SemaphoreType.DMA,
  ]

  @functools.partial(pl.kernel, out_type=out_type, mesh=mesh,
                     scratch_types=scratch)
  def k(*refs):
    ids = refs[0:_NSC]
    tabs = refs[_NSC:2 * _NSC]
    outs = refs[2 * _NSC:3 * _NSC]
    (idx5a, idx5b, idx16, r5a, r5b, r5c, r5d, r16a, r16b, acca, accb,
     g0, g1, g2, g3, g4, g5, oa, ob) = refs[3 * _NSC:]
    wid = lax.axis_index("s") * 2 + lax.axis_index("c")
    base = wid * rpw
    accs = (acca, accb)
    osems = (oa, ob)
    idx5s = (idx5a, idx5b)

    def cfg(fi):
      L = _SC_FEATS[fi][1]
      if L == 16:
        return idx16, (r16a, r16b), (g4, g5), 2, subs[16]
      return (idx5s[(fi - 1) % 2], (r5a, r5b, r5c, r5d), (g0, g1, g2, g3),
              4, subs[5])

    def idx_copy(fi):
      idx_v, _, _, _, SUB = cfg(fi)
      nsub = rpw // SUB
      chunk0 = pl.multiple_of(base // SUB, 8)
      pltpu.sync_copy(ids[fi].at[pl.ds(chunk0, nsub)], idx_v)

    def fire(fi, c, p):
      idx_v, rows, gsems, _, _ = cfg(fi)
      pltpu.async_copy(tabs[fi].at[idx_v.at[c]], rows[p], gsems[p])

    def drain_and_pool(fi, c, p, acc_v):
      idx_v, rows, gsems, _, SUB = cfg(fi)
      L = _SC_FEATS[fi][1]
      scale = 1.0 if L == 16 else 1.0 / L
      pltpu.make_async_copy(tabs[fi].at[idx_v.at[c]], rows[p],
                            gsems[p]).wait()

      def acc_body(b, c2, c=c, L=L, scale=scale, p=p, rows=rows,
                   acc_v=acc_v, SUB=SUB):
        for j in range(_D // 16):
          v = rows[p][b * L, pl.ds(j * 16, 16)]
          for l in range(1, L):
            v = v + rows[p][b * L + l, pl.ds(j * 16, 16)]
          acc_v[c * SUB + b, pl.ds(j * 16, 16)] = v * scale
        return c2

      lax.fori_loop(0, SUB, acc_body, 0)

    idx_copy(0)
    _, _, _, nbuf0, _ = cfg(0)
    for c in range(nbuf0 - 1):
      fire(0, c, c)

    for fi in range(_NSC):
      _, L, _ = _SC_FEATS[fi]
      idx_v, rows, gsems, nbuf, SUB = cfg(fi)
      nsub = rpw // SUB
      acc_v = accs[fi % 2]

      if fi >= 2:
        pltpu.make_async_copy(acc_v, outs[fi - 2].at[pl.ds(base, rpw)],
                              osems[fi % 2]).wait()

      next_cross = fi + 1 < _NSC and _SC_FEATS[fi + 1][1] != L
      nlast = nsub // nbuf - 1

      def step_body(t, carry, fi=fi, nbuf=nbuf, nsub=nsub,
                    next_cross=next_cross, nlast=nlast):
        last = t == nlast
        for kk in range(nbuf):
          c = nbuf * t + kk

          @pl.when(c + nbuf - 1 < nsub)
          def _():
            fire(fi, c + nbuf - 1, (kk + nbuf - 1) % nbuf)

          if kk == 0 and fi + 1 < _NSC:
            @pl.when(last)
            def _():
              idx_copy(fi + 1)
              if next_cross:
                nbuf_n = cfg(fi + 1)[3]
                for cn in range(nbuf_n - 1):
                  fire(fi + 1, cn, cn)

          drain_and_pool(fi, c, kk, accs[fi % 2])

          if fi + 1 < _NSC and not next_cross and kk < nbuf - 1:
            @pl.when(last)
            def _():
              fire(fi + 1, kk, kk)
        return carry

      lax.fori_loop(0, nsub // nbuf, step_body, 0)
      pltpu.async_copy(acc_v, outs[fi].at[pl.ds(base, rpw)],
                       osems[fi % 2])

    for fi in (_NSC - 2, _NSC - 1):
      pltpu.make_async_copy(accs[fi % 2], outs[fi].at[pl.ds(base, rpw)],
                            osems[fi % 2]).wait()

  return k(*flat_ids, *tables)


_VTOT = sum(v for _, v, _ in _TC_FEATS)
_N5 = [s for s, (_, _, L) in enumerate(_TC_FEATS) if L == 5]


def _tc_body(refs, n_small):
  i = 0
  sc_refs = refs[i:i + _NSC]; i += _NSC
  pn_ids_ref = refs[i]; i += 1
  t0_ref = refs[i]; i += 1
  a0_ref = refs[i]; i += 1
  a1_ref = refs[i]; i += 1
  p0_ref = refs[i]; i += 1
  p7_ref = refs[i]; i += 1
  iota_ref = refs[i]; i += 1
  iota5_ref = refs[i]; i += 1
  scale_ref = refs[i]; i += 1
  small_tab_refs = refs[i:i + n_small]; i += n_small
  w0_ref, b0_ref, w1_ref, b1_ref, w2_ref, b2_ref = refs[i:i + 6]
  out_ref = refs[i + 6]
  wf_ref = refs[i + 7]

  pn_ids = pn_ids_ref[...]
  n0 = jnp.sum((pn_ids == 0).astype(jnp.float32), axis=1, keepdims=True)
  x0 = (sc_refs[0][...] - n0 * t0_ref[...]) / jnp.maximum(16.0 - n0, 1.0)

  iotar = iota_ref[...]
  iota5 = iota5_ref[...]
  b0c = jnp.dot(a0_ref[...], p0_ref[...],
                preferred_element_type=jnp.float32)
  cnt = (b0c == iotar).astype(jnp.float32)
  a1 = a1_ref[...]
  for l in range(4):
    blc = jnp.dot(a1[:, l * 7:(l + 1) * 7], p7_ref[...],
                  preferred_element_type=jnp.float32)
    cnt = cnt + (blc == iota5).astype(jnp.float32)
  cnt = cnt * scale_ref[...]

  @pl.when(pl.program_id(0) == 0)
  def _():
    wf_parts = []
    for s, (fi, vocab, L) in enumerate(_TC_FEATS):
      wf_parts.append(jnp.dot(small_tab_refs[s][...],
                              w0_ref[fi * _D:(fi + 1) * _D, :],
                              preferred_element_type=jnp.float32))
    wf_ref[...] = jnp.concatenate(wf_parts, axis=0)

  wf = wf_ref[...]

  x = jnp.concatenate([x0] + [sc_refs[k][...] for k in range(1, _NSC)],
                      axis=1)
  w0sc = jnp.concatenate([w0_ref[0:_D, :], w0_ref[6 * _D:14 * _D, :]],
                         axis=0)

  bf = jnp.bfloat16
  h = (jnp.dot(x.astype(bf), w0sc.astype(bf),
               preferred_element_type=jnp.float32)
       + jnp.dot(cnt.astype(bf), wf.astype(bf),
                 preferred_element_type=jnp.float32))
  h = jnp.maximum(h + b0_ref[...], 0.0)
  h = jnp.dot(h.astype(bf), w1_ref[...].astype(bf),
              preferred_element_type=jnp.float32)
  h = jnp.maximum(h + b1_ref[...], 0.0)
  h = jnp.dot(h.astype(bf), w2_ref[...].astype(bf),
              preferred_element_type=jnp.float32)
  out_ref[...] = h + b2_ref[...]


def _tc_mlp(sc_feats, pn_ids, t0, a0, a1, p0, p7, iota_row, iota5_row,
            scale_row, small_tabs, weights, interpret=False):
  n_small = len(small_tabs)
  nb = sc_feats[0].shape[0]
  rows = 512
  grid = (nb // rows,)

  def rowblk(cols):
    return pl.BlockSpec((rows, cols), lambda i: (i, 0))

  def full(a):
    return pl.BlockSpec(a.shape, lambda i: (0, 0))

  in_specs = (
      [rowblk(_D) for _ in sc_feats]
      + [rowblk(16)]
      + [full(t0)]
      + [rowblk(12), rowblk(28), full(p0), full(p7), full(iota_row),
         full(iota5_row), full(scale_row)]
      + [full(t) for t in small_tabs]
      + [full(w) for w in weights]
  )
  out_spec = pl.BlockSpec((rows, _D), lambda i: (i, 0))

  f = pl.pallas_call(
      lambda *refs: _tc_body(refs, n_small),
      grid=grid,
      in_specs=in_specs,
      out_specs=out_spec,
      out_shape=jax.ShapeDtypeStruct((nb, _D), jnp.float32),
      scratch_shapes=[pltpu.VMEM((_VTOT, 512), jnp.float32)],
      interpret=interpret,
  )
  return f(*sc_feats, pn_ids, t0, a0, a1, p0, p7, iota_row, iota5_row,
           scale_row, *small_tabs, *weights)


_FEAT_NAMES = (
    "pl_name", "pl_collab", "pl_duration", "num_pl_songs", "num_pl_artists",
    "num_pl_albums", "track_uri", "track_name", "artist_uri", "artist_name",
    "album_uri", "album_name", "artist_genres", "pl_titles", "duration_songs",
    "track_pop", "artist_pop", "artists_followers", "track_danceability",
    "track_energy", "track_key",
)


def kernel(params, pl_name_ids, pl_collab_ids, pl_duration_ids,
           num_pl_songs_ids, num_pl_artists_ids, num_pl_albums_ids,
           track_uri_ids, track_name_ids, artist_uri_ids, artist_name_ids,
           album_uri_ids, album_name_ids, artist_genres_ids, pl_titles_ids,
           duration_songs_ids, track_pop_ids, artist_pop_ids,
           artists_followers_ids, track_danceability_ids, track_energy_ids,
           track_key_ids):
  ids_by_index = (
      pl_name_ids, pl_collab_ids, pl_duration_ids, num_pl_songs_ids,
      num_pl_artists_ids, num_pl_albums_ids, track_uri_ids, track_name_ids,
      artist_uri_ids, artist_name_ids, album_uri_ids, album_name_ids,
      artist_genres_ids, pl_titles_ids, duration_songs_ids, track_pop_ids,
      artist_pop_ids, artists_followers_ids, track_danceability_ids,
      track_energy_ids, track_key_ids,
  )

  sc_tabs = [params["t_" + _FEAT_NAMES[fi]] for fi, _, _ in _SC_FEATS]

  _NH = 1
  nb = _B // _NH
  rpw = nb // _NW
  sc_half_ids = []
  for h in range(_NH):
    lo = h * nb
    ids_h = []
    for fi, L, _ in _SC_FEATS:
      cnt = L * (rpw // 8 if L == 5 else rpw // 16)
      ids_h.append(ids_by_index[fi][lo:lo + nb].reshape(-1, cnt))
    sc_half_ids.append(ids_h)
  sc_feats_halves = [_sc_gather_pool(sc_tabs, sc_half_ids[h], nb)
                     for h in range(_NH)]

  small_ids = [ids_by_index[fi].reshape(_B, -1) for fi, _, _ in _TC_FEATS]
  small_tabs = [params["t_" + _FEAT_NAMES[fi]] for fi, _, _ in _TC_FEATS]
  t0 = params["t_pl_name"][0:1]

  a0 = jnp.stack([ids.astype(jnp.float32)[:, 0] for ids in small_ids],
                 axis=1)
  a1 = jnp.concatenate(
      [jnp.stack([small_ids[s].astype(jnp.float32)[:, l] for s in _N5],
                 axis=1) for l in range(1, 5)], axis=1)
  offs = np.cumsum([0] + [v for _, v, _ in _TC_FEATS])
  p0 = np.zeros((12, _VTOT), np.float32)
  for s in range(len(_TC_FEATS)):
    p0[s, offs[s]:offs[s + 1]] = 1.0
  p7 = np.zeros((7, _VTOT), np.float32)
  for j, s in enumerate(_N5):
    p7[j, offs[s]:offs[s + 1]] = 1.0
  iota_row = np.concatenate(
      [np.arange(v, dtype=np.float32) for _, v, _ in _TC_FEATS])[None]
  iota5_row = np.concatenate(
      [np.arange(v, dtype=np.float32) if L == 5 else np.full(v, -1.0,
                                                             np.float32)
       for _, v, L in _TC_FEATS])[None]
  scale_row = np.concatenate(
      [np.full(v, 1.0 if L == 1 else 1.0 / L, np.float32)
       for _, v, L in _TC_FEATS])[None]

  weights = [
      params["W0"], params["b0"].reshape(1, -1),
      params["W1"], params["b1"].reshape(1, -1),
      params["W2"], params["b2"].reshape(1, -1),
  ]
  outs = []
  for h in range(_NH):
    lo = h * nb
    outs.append(_tc_mlp(
        sc_feats_halves[h], pl_name_ids[lo:lo + nb], t0,
        a0[lo:lo + nb], a1[lo:lo + nb], jnp.asarray(p0), jnp.asarray(p7),
        jnp.asarray(iota_row), jnp.asarray(iota5_row),
        jnp.asarray(scale_row), small_tabs, weights))
  return jnp.concatenate(outs, axis=0)
